# baseline (speedup 1.0000x reference)
import jax
import jax.numpy as jnp
from jax import lax
from jax.experimental import pallas as pl
from jax.experimental.pallas import tpu as pltpu
from jax.experimental.pallas import tpu_sc as plsc

_B = 16384
_DIM = 64
_NC = 2
_NS = 16
_LANES = 16
_NW = _NC * _NS
_BPW = _B // _NW
_RND = 128
_NRND = _BPW // _RND


def _permute16(x, idx):
    dn = lax.GatherDimensionNumbers(
        offset_dims=(), collapsed_slice_dims=(0,), start_index_map=(0,))
    return lax.gather(x, idx[:, None], dn, (1,),
                      mode=lax.GatherScatterMode.PROMISE_IN_BOUNDS)


def _sqrt16(x):
    y = jnp.where(x > 4096.0, 0.001 * x + 64.0, 0.0625 * x + 4.0)
    for _ in range(7):
        y = 0.5 * (y + x / y)
    return jnp.where(x > 0.0, y, 0.0)


def _body(hpar_hbm, rpar_hbm, tpar_hbm, hp_hbm, rp_hbm, tp_hbm,
          ent_hbm, relt_hbm, out_hbm,
          hpv, rpv, tpv, hiv, riv, tiv, h_buf, r_buf, t_buf,
          scores_v, sem):
    wid = lax.axis_index("s") * _NC + lax.axis_index("c")
    base = wid * _BPW

    lanes = jnp.arange(_LANES, dtype=jnp.int32)

    for rnd in range(_NRND):
        roff = base + rnd * _RND
        pltpu.sync_copy(hpar_hbm.at[pl.ds(roff, _RND)], hpv)
        pltpu.sync_copy(rpar_hbm.at[pl.ds(roff, _RND)], rpv)
        pltpu.sync_copy(tpar_hbm.at[pl.ds(roff, _RND)], tpv)
        pltpu.sync_copy(hp_hbm.at[pl.ds(roff, _RND)], hiv)
        pltpu.sync_copy(rp_hbm.at[pl.ds(roff, _RND)], riv)
        pltpu.sync_copy(tp_hbm.at[pl.ds(roff, _RND)], tiv)

        ch = pltpu.async_copy(ent_hbm.at[hiv], h_buf, sem)
        cr = pltpu.async_copy(relt_hbm.at[riv], r_buf, sem)
        ct = pltpu.async_copy(ent_hbm.at[tiv], t_buf, sem)
        ch.wait()
        cr.wait()
        ct.wait()

        def grp_body(g, carry):
            gsl = pl.ds(g * _LANES, _LANES)
            hp16 = hpv[gsl]
            rp16 = rpv[gsl]
            tp16 = tpv[gsl]
            vecs = []
            for l in range(_LANES):
                i = g * _LANES + l
                bcast = jnp.full((_LANES,), l, jnp.int32)
                ph = _permute16(hp16, bcast)
                pr = _permute16(rp16, bcast)
                pt = _permute16(tp16, bcast)
                acc = jnp.zeros((_LANES,), jnp.float32)
                for j in range(_DIM // _LANES):
                    s0 = pl.ds(j * _LANES, _LANES)
                    s1 = pl.ds(_DIM + j * _LANES, _LANES)
                    h0 = h_buf[i, s0]
                    r0 = r_buf[i, s0]
                    t0 = t_buf[i, s0]
                    hv = h0 + ph * (h_buf[i, s1] - h0)
                    rv = r0 + pr * (r_buf[i, s1] - r0)
                    tv = t0 + pt * (t_buf[i, s1] - t0)
                    d = (hv + rv) - tv
                    acc = acc + d * d
                vecs.append(acc)
            for s in range(4):
                step = 1 << s
                folded = [v + _permute16(v, lanes ^ step) for v in vecs]
                bit = lax.bitwise_and(lax.shift_right_logical(lanes, s), 1)
                vecs = [jnp.where(bit == 0, folded[2 * k], folded[2 * k + 1])
                        for k in range(len(folded) // 2)]
            scores_v[pl.ds(rnd * _RND + g * _LANES, _LANES)] = _sqrt16(vecs[0])
            return carry

        lax.fori_loop(0, _RND // _LANES, grp_body, 0)

    pltpu.sync_copy(scores_v, out_hbm.at[pl.ds(base, _BPW)])


_mesh = plsc.VectorSubcoreMesh(core_axis_name="c", subcore_axis_name="s")

_kernel_call = pl.kernel(
    _body,
    out_type=jax.ShapeDtypeStruct((_B,), jnp.float32),
    scratch_types=[
        pltpu.VMEM((_RND,), jnp.float32),
        pltpu.VMEM((_RND,), jnp.float32),
        pltpu.VMEM((_RND,), jnp.float32),
        pltpu.VMEM((_RND,), jnp.int32),
        pltpu.VMEM((_RND,), jnp.int32),
        pltpu.VMEM((_RND,), jnp.int32),
        pltpu.VMEM((_RND, 2 * _DIM), jnp.float32),
        pltpu.VMEM((_RND, 2 * _DIM), jnp.float32),
        pltpu.VMEM((_RND, 2 * _DIM), jnp.float32),
        pltpu.VMEM((_BPW,), jnp.float32),
        pltpu.SemaphoreType.DMA,
    ],
    mesh=_mesh,
)


@jax.jit
def kernel(heads, rels, tails, entity_table, rel_table):
    ent2 = entity_table.reshape(-1, 2 * _DIM)
    rel2 = rel_table.reshape(-1, 2 * _DIM)
    return _kernel_call(lax.bitwise_and(heads, 1).astype(jnp.float32),
                        lax.bitwise_and(rels, 1).astype(jnp.float32),
                        lax.bitwise_and(tails, 1).astype(jnp.float32),
                        lax.shift_right_logical(heads, 1),
                        lax.shift_right_logical(rels, 1),
                        lax.shift_right_logical(tails, 1),
                        ent2, rel2)
